# baseline (speedup 1.0000x reference)
import functools

import jax
import jax.numpy as jnp
from jax import lax
from jax.experimental import pallas as pl
from jax.experimental.pallas import tpu as pltpu
from jax.experimental.pallas import tpu_sc as plsc

B, S, D = 2, 2048, 2048
DFF = 4 * D
BS = B * S
MIN_CAP, MAX_CAP = 0.25, 1.0

_T = 512
_F = 1024
_NT = BS // _T
_NF = DFF // _F

_NC = 2
_NS = 16
_NW = _NC * _NS
_CH = BS // _NW
_SUB = 32
_NSUB = _CH // _SUB


def _gelu_exact(x):
    return 0.5 * x * (1.0 + jax.lax.erf(x * (2.0 ** -0.5)))


def _head_kernel(hid_ref, w1_ref, b1_ref, w2_ref, b2_ref, rw_ref, rb_ref,
                 w_ref, k_ref, gidx_ref):
    hid = hid_ref[...]
    pooled = jnp.mean(hid, axis=1)
    h1 = jnp.dot(pooled.astype(jnp.bfloat16), w1_ref[...].astype(jnp.bfloat16),
                 preferred_element_type=jnp.float32) + b1_ref[...]
    h1 = _gelu_exact(h1)
    c = jax.nn.sigmoid(jnp.dot(h1.astype(jnp.bfloat16),
                               w2_ref[...].astype(jnp.bfloat16),
                               preferred_element_type=jnp.float32) + b2_ref[...])
    cap = MIN_CAP + jnp.mean(c) * (MAX_CAP - MIN_CAP)
    k = (cap * S).astype(jnp.int32)
    k_ref[0] = k

    logits = jnp.dot(jnp.reshape(hid, (BS, D)).astype(jnp.bfloat16),
                     jnp.reshape(rw_ref[...], (D, 1)).astype(jnp.bfloat16),
                     preferred_element_type=jnp.float32)
    logits = jnp.reshape(logits, (B, S)) + rb_ref[0, 0]

    keys = jax.lax.bitcast_convert_type(logits, jnp.int32)
    keys = jnp.where(keys >= 0, keys, keys ^ jnp.int32(0x7FFFFFFF))
    thr = jnp.full((B, 1), -2147483647 - 1, jnp.int32)
    for bit in range(30, -1, -1):
        cand = thr + jnp.int32(1 << bit)
        cnt = jnp.sum((keys >= cand).astype(jnp.int32), axis=1, keepdims=True)
        thr = jnp.where(cnt >= k, cand, thr)

    ltm = (jax.lax.broadcasted_iota(jnp.int32, (S, S), 0)
           < jax.lax.broadcasted_iota(jnp.int32, (S, S), 1)).astype(jnp.bfloat16)
    gt = keys > thr
    eq = keys == thr
    n_gt = jnp.sum(gt.astype(jnp.int32), axis=1, keepdims=True)
    eq_excl = jnp.dot(eq.astype(jnp.bfloat16), ltm,
                      preferred_element_type=jnp.float32).astype(jnp.int32)
    sel = gt | (eq & (eq_excl < (k - n_gt)))
    w_ref[...] = jnp.where(sel, jax.nn.sigmoid(logits), 0.0)

    pos = jnp.dot(sel.astype(jnp.bfloat16), ltm,
                  preferred_element_type=jnp.float32).astype(jnp.int32)
    iota_t = jax.lax.broadcasted_iota(jnp.int32, (B, S), 1)
    perm = jnp.where(sel, pos, k + iota_t - pos)
    gidx_ref[...] = perm + jax.lax.broadcasted_iota(jnp.int32, (B, S), 0) * S


def _disperse_kernel(hid_ref, gidx_ref, w_ref, xc_ref, wc_ref,
                     idx_v, rows_v, wv_v, sem):
    wid = lax.axis_index("s") * _NC + lax.axis_index("c")
    base = wid * _CH
    pltpu.sync_copy(gidx_ref.at[wid], idx_v)
    pltpu.sync_copy(w_ref.at[wid], wv_v)
    for j in range(_NSUB):
        pltpu.sync_copy(hid_ref.at[pl.ds(base + j * _SUB, _SUB)], rows_v)
        pltpu.async_copy(rows_v, xc_ref.at[idx_v.at[j]], sem).wait()
        pltpu.async_copy(wv_v.at[j], wc_ref.at[idx_v.at[j]], sem).wait()


def _merge_kernel(yc_ref, gidx_ref, out_ref, idx_v, rows_v, sem):
    wid = lax.axis_index("s") * _NC + lax.axis_index("c")
    base = wid * _CH
    pltpu.sync_copy(gidx_ref.at[wid], idx_v)
    for j in range(_NSUB):
        pltpu.async_copy(yc_ref.at[idx_v.at[j]], rows_v, sem).wait()
        pltpu.sync_copy(rows_v, out_ref.at[pl.ds(base + j * _SUB, _SUB)])


def _ffn_kernel(kk_ref, x_ref, w_ref, wf1_ref, bf1_ref, wf2_ref, bf2_ref,
                out_ref, acc_ref):
    t = pl.program_id(0)
    f = pl.program_id(1)
    valid = lax.rem(t * _T, S) < kk_ref[0]

    @pl.when(valid)
    def _():
        x = x_ref[...]
        h = jnp.dot(x.astype(jnp.bfloat16), wf1_ref[...],
                    preferred_element_type=jnp.float32) + bf1_ref[...]
        h = _gelu_exact(h)
        p = jnp.dot(h.astype(jnp.bfloat16), wf2_ref[...],
                    preferred_element_type=jnp.float32)

        @pl.when(f == 0)
        def _():
            acc_ref[...] = p

        @pl.when(f > 0)
        def _():
            acc_ref[...] += p

        @pl.when(f == _NF - 1)
        def _():
            out_ref[...] = x + w_ref[...] * (acc_ref[...] + bf2_ref[...])

    @pl.when(jnp.logical_not(valid) & (f == _NF - 1))
    def _():
        out_ref[...] = x_ref[...]


def _valid_t(t, kk):
    return lax.rem(t * _T, S) < kk[0]


def kernel(hidden_states, W1, b1, W2, b2, router_weight, router_bias,
           Wf1, bf1, Wf2, bf2):
    w, kk, gidx = pl.pallas_call(
        _head_kernel,
        out_shape=[
            jax.ShapeDtypeStruct((B, S), jnp.float32),
            jax.ShapeDtypeStruct((1,), jnp.int32),
            jax.ShapeDtypeStruct((B, S), jnp.int32),
        ],
        out_specs=[
            pl.BlockSpec(memory_space=pltpu.VMEM),
            pl.BlockSpec(memory_space=pltpu.SMEM),
            pl.BlockSpec(memory_space=pltpu.VMEM),
        ],
    )(hidden_states, W1, b1.reshape(1, D // 4), W2, b2.reshape(1, 1),
      router_weight.reshape(1, D), router_bias.reshape(1, 1))

    mesh = plsc.VectorSubcoreMesh(core_axis_name="c", subcore_axis_name="s")
    xc, wc = pl.kernel(
        _disperse_kernel,
        mesh=mesh,
        out_type=[
            jax.ShapeDtypeStruct((BS, D), jnp.float32),
            jax.ShapeDtypeStruct((BS,), jnp.float32),
        ],
        scratch_types=[
            pltpu.VMEM((_NSUB, _SUB), jnp.int32),
            pltpu.VMEM((_SUB, D), jnp.float32),
            pltpu.VMEM((_NSUB, _SUB), jnp.float32),
            pltpu.SemaphoreType.DMA,
        ],
    )(hidden_states.reshape(BS, D), gidx.reshape(_NW, _NSUB, _SUB),
      w.reshape(_NW, _NSUB, _SUB))

    yc = pl.pallas_call(
        _ffn_kernel,
        grid_spec=pltpu.PrefetchScalarGridSpec(
            num_scalar_prefetch=1,
            grid=(_NT, _NF),
            in_specs=[
                pl.BlockSpec((_T, D), lambda t, f, kk: (t, 0)),
                pl.BlockSpec((_T, 1), lambda t, f, kk: (t, 0)),
                pl.BlockSpec((D, _F),
                             lambda t, f, kk: (0, jnp.where(_valid_t(t, kk), f, 0))),
                pl.BlockSpec((1, _F),
                             lambda t, f, kk: (0, jnp.where(_valid_t(t, kk), f, 0))),
                pl.BlockSpec((_F, D),
                             lambda t, f, kk: (jnp.where(_valid_t(t, kk), f, 0), 0)),
                pl.BlockSpec((1, D), lambda t, f, kk: (0, 0)),
            ],
            out_specs=pl.BlockSpec((_T, D), lambda t, f, kk: (t, 0)),
            scratch_shapes=[pltpu.VMEM((_T, D), jnp.float32)],
        ),
        out_shape=jax.ShapeDtypeStruct((BS, D), jnp.float32),
        compiler_params=pltpu.CompilerParams(
            dimension_semantics=("arbitrary", "arbitrary")),
    )(kk, xc, wc.reshape(BS, 1),
      Wf1.astype(jnp.bfloat16), bf1.reshape(1, DFF),
      Wf2.astype(jnp.bfloat16), bf2.reshape(1, D))

    out = pl.kernel(
        _merge_kernel,
        mesh=mesh,
        out_type=jax.ShapeDtypeStruct((BS, D), jnp.float32),
        scratch_types=[
            pltpu.VMEM((_NSUB, _SUB), jnp.int32),
            pltpu.VMEM((_SUB, D), jnp.float32),
            pltpu.SemaphoreType.DMA,
        ],
    )(yc, gidx.reshape(_NW, _NSUB, _SUB))

    return out.reshape(B, S, D)


# speedup vs baseline: 1.0473x; 1.0473x over previous
import functools

import jax
import jax.numpy as jnp
from jax import lax
from jax.experimental import pallas as pl
from jax.experimental.pallas import tpu as pltpu
from jax.experimental.pallas import tpu_sc as plsc

B, S, D = 2, 2048, 2048
DFF = 4 * D
BS = B * S
MIN_CAP, MAX_CAP = 0.25, 1.0

_T = 512
_F = 1024
_NT = BS // _T
_NF = DFF // _F

_NC = 2
_NS = 16
_NW = _NC * _NS
_CH = BS // _NW
_SUB = 16
_NSUB = _CH // _SUB
_WREP = 128


def _gelu_exact(x):
    return 0.5 * x * (1.0 + jax.lax.erf(x * (2.0 ** -0.5)))


def _head_kernel(hid_ref, w1_ref, b1_ref, w2_ref, b2_ref, rw_ref, rb_ref,
                 w_ref, k_ref, gidx_ref):
    hid = hid_ref[...]
    pooled = jnp.mean(hid, axis=1)
    h1 = jnp.dot(pooled.astype(jnp.bfloat16), w1_ref[...].astype(jnp.bfloat16),
                 preferred_element_type=jnp.float32) + b1_ref[...]
    h1 = _gelu_exact(h1)
    c = jax.nn.sigmoid(jnp.dot(h1.astype(jnp.bfloat16),
                               w2_ref[...].astype(jnp.bfloat16),
                               preferred_element_type=jnp.float32) + b2_ref[...])
    cap = MIN_CAP + jnp.mean(c) * (MAX_CAP - MIN_CAP)
    k = (cap * S).astype(jnp.int32)
    k_ref[0] = k

    logits = jnp.dot(jnp.reshape(hid, (BS, D)).astype(jnp.bfloat16),
                     jnp.reshape(rw_ref[...], (D, 1)).astype(jnp.bfloat16),
                     preferred_element_type=jnp.float32)
    logits = jnp.reshape(logits, (B, S)) + rb_ref[0, 0]

    keys = jax.lax.bitcast_convert_type(logits, jnp.int32)
    keys = jnp.where(keys >= 0, keys, keys ^ jnp.int32(0x7FFFFFFF))
    thr = jnp.full((B, 1), -2147483647 - 1, jnp.int32)
    for bit in range(30, -1, -1):
        cand = thr + jnp.int32(1 << bit)
        cnt = jnp.sum((keys >= cand).astype(jnp.int32), axis=1, keepdims=True)
        thr = jnp.where(cnt >= k, cand, thr)

    ltm = (jax.lax.broadcasted_iota(jnp.int32, (S, S), 0)
           < jax.lax.broadcasted_iota(jnp.int32, (S, S), 1)).astype(jnp.bfloat16)
    gt = keys > thr
    eq = keys == thr
    n_gt = jnp.sum(gt.astype(jnp.int32), axis=1, keepdims=True)
    eq_excl = jnp.dot(eq.astype(jnp.bfloat16), ltm,
                      preferred_element_type=jnp.float32).astype(jnp.int32)
    sel = gt | (eq & (eq_excl < (k - n_gt)))
    w_ref[...] = jnp.where(sel, jax.nn.sigmoid(logits), 0.0)

    pos = jnp.dot(sel.astype(jnp.bfloat16), ltm,
                  preferred_element_type=jnp.float32).astype(jnp.int32)
    iota_t = jax.lax.broadcasted_iota(jnp.int32, (B, S), 1)
    perm = jnp.where(sel, pos, k + iota_t - pos)
    gidx_ref[...] = perm + jax.lax.broadcasted_iota(jnp.int32, (B, S), 0) * S


def _disperse_kernel(hid_ref, gidx_ref, w_ref, xc_ref, wc_ref,
                     idx_v, rows_v, wv_v, sem_r, sem_x, sem_w):
    wid = lax.axis_index("s") * _NC + lax.axis_index("c")
    base = wid * _CH
    pltpu.sync_copy(gidx_ref.at[wid], idx_v)
    pltpu.sync_copy(w_ref.at[wid], wv_v)
    rd = [None] * _NSUB
    sx = [None] * _NSUB
    rd[0] = pltpu.async_copy(hid_ref.at[pl.ds(base, _SUB)], rows_v.at[0],
                             sem_r.at[0])
    for j in range(_NSUB):
        if j + 1 < _NSUB:
            if j >= 1:
                sx[j - 1].wait()
            rd[j + 1] = pltpu.async_copy(
                hid_ref.at[pl.ds(base + (j + 1) * _SUB, _SUB)],
                rows_v.at[(j + 1) % 2], sem_r.at[(j + 1) % 2])
        rd[j].wait()
        sx[j] = pltpu.async_copy(rows_v.at[j % 2], xc_ref.at[idx_v.at[j]], sem_x)
        pltpu.async_copy(wv_v.at[j], wc_ref.at[idx_v.at[j]], sem_w).wait()
    sx[_NSUB - 1].wait()


def _merge_kernel(yc_ref, gidx_ref, out_ref, idx_v, rows_v, sem_g, sem_o):
    wid = lax.axis_index("s") * _NC + lax.axis_index("c")
    base = wid * _CH
    pltpu.sync_copy(gidx_ref.at[wid], idx_v)
    g = [None] * _NSUB
    wr = [None] * _NSUB
    g[0] = pltpu.async_copy(yc_ref.at[idx_v.at[0]], rows_v.at[0], sem_g.at[0])
    for j in range(_NSUB):
        if j + 1 < _NSUB:
            if j >= 1:
                wr[j - 1].wait()
            g[j + 1] = pltpu.async_copy(yc_ref.at[idx_v.at[j + 1]],
                                        rows_v.at[(j + 1) % 2],
                                        sem_g.at[(j + 1) % 2])
        g[j].wait()
        wr[j] = pltpu.async_copy(rows_v.at[j % 2],
                                 out_ref.at[pl.ds(base + j * _SUB, _SUB)], sem_o)
    wr[_NSUB - 1].wait()


def _ffn_kernel(kk_ref, x_ref, w_ref, wf1_ref, bf1_ref, wf2_ref, bf2_ref,
                out_ref, acc_ref):
    t = pl.program_id(0)
    f = pl.program_id(1)
    valid = lax.rem(t * _T, S) < kk_ref[0]

    @pl.when(valid)
    def _():
        x = x_ref[...]
        h = jnp.dot(x.astype(jnp.bfloat16), wf1_ref[...],
                    preferred_element_type=jnp.float32) + bf1_ref[...]
        h = _gelu_exact(h)
        p = jnp.dot(h.astype(jnp.bfloat16), wf2_ref[...],
                    preferred_element_type=jnp.float32)

        @pl.when(f == 0)
        def _():
            acc_ref[...] = p

        @pl.when(f > 0)
        def _():
            acc_ref[...] += p

        @pl.when(f == _NF - 1)
        def _():
            out_ref[...] = x + w_ref[...][:, 0:1] * (acc_ref[...] + bf2_ref[...])

    @pl.when(jnp.logical_not(valid) & (f == _NF - 1))
    def _():
        out_ref[...] = x_ref[...]


def _valid_t(t, kk):
    return lax.rem(t * _T, S) < kk[0]


def kernel(hidden_states, W1, b1, W2, b2, router_weight, router_bias,
           Wf1, bf1, Wf2, bf2):
    w, kk, gidx = pl.pallas_call(
        _head_kernel,
        out_shape=[
            jax.ShapeDtypeStruct((B, S), jnp.float32),
            jax.ShapeDtypeStruct((1,), jnp.int32),
            jax.ShapeDtypeStruct((B, S), jnp.int32),
        ],
        out_specs=[
            pl.BlockSpec(memory_space=pltpu.VMEM),
            pl.BlockSpec(memory_space=pltpu.SMEM),
            pl.BlockSpec(memory_space=pltpu.VMEM),
        ],
    )(hidden_states, W1, b1.reshape(1, D // 4), W2, b2.reshape(1, 1),
      router_weight.reshape(1, D), router_bias.reshape(1, 1))

    mesh = plsc.VectorSubcoreMesh(core_axis_name="c", subcore_axis_name="s")
    w16 = jnp.broadcast_to(w.reshape(BS, 1), (BS, _WREP))
    xc, wc = pl.kernel(
        _disperse_kernel,
        mesh=mesh,
        out_type=[
            jax.ShapeDtypeStruct((BS, D), jnp.float32),
            jax.ShapeDtypeStruct((BS, _WREP), jnp.float32),
        ],
        scratch_types=[
            pltpu.VMEM((_NSUB, _SUB), jnp.int32),
            pltpu.VMEM((2, _SUB, D), jnp.float32),
            pltpu.VMEM((_NSUB, _SUB, _WREP), jnp.float32),
            pltpu.SemaphoreType.DMA((2,)),
            pltpu.SemaphoreType.DMA,
            pltpu.SemaphoreType.DMA,
        ],
    )(hidden_states.reshape(BS, D), gidx.reshape(_NW, _NSUB, _SUB),
      w16.reshape(_NW, _NSUB, _SUB, _WREP))

    yc = pl.pallas_call(
        _ffn_kernel,
        grid_spec=pltpu.PrefetchScalarGridSpec(
            num_scalar_prefetch=1,
            grid=(_NT, _NF),
            in_specs=[
                pl.BlockSpec((_T, D), lambda t, f, kk: (t, 0)),
                pl.BlockSpec((_T, _WREP), lambda t, f, kk: (t, 0)),
                pl.BlockSpec((D, _F),
                             lambda t, f, kk: (0, jnp.where(_valid_t(t, kk), f, 0))),
                pl.BlockSpec((1, _F),
                             lambda t, f, kk: (0, jnp.where(_valid_t(t, kk), f, 0))),
                pl.BlockSpec((_F, D),
                             lambda t, f, kk: (jnp.where(_valid_t(t, kk), f, 0), 0)),
                pl.BlockSpec((1, D), lambda t, f, kk: (0, 0)),
            ],
            out_specs=pl.BlockSpec((_T, D), lambda t, f, kk: (t, 0)),
            scratch_shapes=[pltpu.VMEM((_T, D), jnp.float32)],
        ),
        out_shape=jax.ShapeDtypeStruct((BS, D), jnp.float32),
        compiler_params=pltpu.CompilerParams(
            dimension_semantics=("arbitrary", "arbitrary")),
    )(kk, xc, wc,
      Wf1.astype(jnp.bfloat16), bf1.reshape(1, DFF),
      Wf2.astype(jnp.bfloat16), bf2.reshape(1, D))

    out = pl.kernel(
        _merge_kernel,
        mesh=mesh,
        out_type=jax.ShapeDtypeStruct((BS, D), jnp.float32),
        scratch_types=[
            pltpu.VMEM((_NSUB, _SUB), jnp.int32),
            pltpu.VMEM((2, _SUB, D), jnp.float32),
            pltpu.SemaphoreType.DMA((2,)),
            pltpu.SemaphoreType.DMA,
        ],
    )(yc, gidx.reshape(_NW, _NSUB, _SUB))

    return out.reshape(B, S, D)
